# baseline (speedup 1.0000x reference)
import functools

import jax
import jax.numpy as jnp
from jax import lax
from jax.experimental import pallas as pl
from jax.experimental.pallas import tpu as pltpu
from jax.experimental.pallas import tpu_sc as plsc

_N = 10000
_E = 320000
_D = 64
_SCALE = 1.0 / (_D ** 0.5)
_EPS = 1e-9

_NCORE = 2
_NSUB = 16
_NW = _NCORE * _NSUB
_EW = _E // _NW
_K = 100
_NCHUNK = _EW // _K
_NBUF = 2
_RPT = 632
_NP = _RPT * _NSUB

_ROWBLK = 1000


def _make_edge_pass():
  mesh = plsc.VectorSubcoreMesh(core_axis_name="c", subcore_axis_name="s",
                                num_cores=_NCORE, num_subcores=_NSUB)

  @functools.partial(
      pl.kernel,
      out_type=(
          jax.ShapeDtypeStruct((_NCORE, _NP, _D), jnp.float32),
          jax.ShapeDtypeStruct((_NCORE, _NP, 16), jnp.float32),
      ),
      mesh=mesh,
      scratch_types=[
          pltpu.VMEM_SHARED((_NP, _D), jnp.float32),
          pltpu.VMEM_SHARED((_NP, 16), jnp.float32),
          pltpu.VMEM((_NCHUNK, _K), jnp.int32),
          pltpu.VMEM((_NCHUNK, _K), jnp.int32),
          [pltpu.VMEM((_K, _D), jnp.float32)] * _NBUF,
          [pltpu.VMEM((_K, _D), jnp.float32)] * _NBUF,
          [pltpu.VMEM((_K, _D), jnp.float32)] * _NBUF,
          [pltpu.VMEM((_K, 16), jnp.float32)] * _NBUF,
          [pltpu.SemaphoreType.DMA] * _NBUF,
      ],
      compiler_params=pltpu.CompilerParams(use_tc_tiling_on_sc=False),
  )
  def kern(table_hbm, src_hbm, dst_hbm, zu_hbm, zs_hbm, u_hbm, s_hbm,
           u_sh, s_sh, sidx_v, didx_v, sbufs, dbufs, mbufs, wbufs, gsems):
    c = lax.axis_index("c")
    t = lax.axis_index("s")
    wid = c * _NSUB + t

    pltpu.sync_copy(src_hbm.at[wid], sidx_v)
    pltpu.sync_copy(dst_hbm.at[wid], didx_v)
    r0 = t * _RPT
    pltpu.sync_copy(zu_hbm.at[pl.ds(r0, _RPT)], u_sh.at[pl.ds(r0, _RPT)])
    pltpu.sync_copy(zs_hbm.at[pl.ds(r0, _RPT)], s_sh.at[pl.ds(r0, _RPT)])
    plsc.subcore_barrier()

    lanes = lax.iota(jnp.int32, 16)

    def allsum(v):
      for step in (8, 4, 2, 1):
        v = v + v.at[jnp.bitwise_xor(lanes, step)].get(
            mode="promise_in_bounds")
      return v

    def gather_issue(j, b):
      pltpu.async_copy(table_hbm.at[sidx_v.at[j]], sbufs[b], gsems[b])
      pltpu.async_copy(table_hbm.at[didx_v.at[j]], dbufs[b], gsems[b])

    def gather_wait(b):
      pltpu.make_async_copy(table_hbm.at[sidx_v.at[0]], sbufs[b],
                            gsems[b]).wait()
      pltpu.make_async_copy(table_hbm.at[didx_v.at[0]], dbufs[b],
                            gsems[b]).wait()

    def make_edge_body(sb, db, mb, wb):
      def edge_body(e):
        acc = jnp.zeros((16,), jnp.float32)
        for q in range(_D // 16):
          acc = acc + sb[e, pl.ds(q * 16, 16)] * db[e, pl.ds(q * 16, 16)]
        wv = jnp.exp(allsum(acc) * _SCALE)
        for q in range(_D // 16):
          mb[e, pl.ds(q * 16, 16)] = sb[e, pl.ds(q * 16, 16)] * wv
        wb[e] = jnp.where(lanes == 0, wv, 0.0)
      return edge_body

    for b in range(_NBUF):
      gather_issue(b, b)

    def step(s2, carry):
      for b in range(_NBUF):
        j = s2 * _NBUF + b
        gather_wait(b)
        plsc.parallel_loop(0, _K, 1, unroll=8)(
            make_edge_body(sbufs[b], dbufs[b], mbufs[b], wbufs[b]))
        idxd = didx_v.at[j]
        pltpu.sync_copy(mbufs[b], u_sh.at[idxd], add=True)
        pltpu.sync_copy(wbufs[b], s_sh.at[idxd], add=True)
        gather_issue(jnp.minimum(j + _NBUF, _NCHUNK - 1), b)
      return carry

    lax.fori_loop(0, _NCHUNK // _NBUF, step, 0)
    for b in range(_NBUF):
      gather_wait(b)

    plsc.subcore_barrier()
    pltpu.sync_copy(u_sh.at[pl.ds(r0, _RPT)], u_hbm.at[c, pl.ds(r0, _RPT)])
    pltpu.sync_copy(s_sh.at[pl.ds(r0, _RPT)], s_hbm.at[c, pl.ds(r0, _RPT)])

  return kern


_edge_pass = _make_edge_pass()


def _mm1_body(x_ref, w_ref, o0_ref, o1_ref, o2_ref, o3_ref):
  ft = jnp.dot(x_ref[...], w_ref[...], preferred_element_type=jnp.float32)
  o0_ref[...] = ft[:, 0:64]
  o1_ref[...] = ft[:, 64:128]
  o2_ref[...] = ft[:, 128:192]
  o3_ref[...] = ft[:, 192:256]


def _mm1(x, W1):
  grid = _N // _ROWBLK
  ospec = pl.BlockSpec((_ROWBLK, 64), lambda i: (i, 0))
  oshape = jax.ShapeDtypeStruct((_N, 64), jnp.float32)
  return pl.pallas_call(
      _mm1_body,
      grid=(grid,),
      in_specs=[
          pl.BlockSpec((_ROWBLK, 128), lambda i: (i, 0)),
          pl.BlockSpec((128, 256), lambda i: (0, 0)),
      ],
      out_specs=(ospec, ospec, ospec, ospec),
      out_shape=(oshape, oshape, oshape, oshape),
  )(x, W1)


def _head_out(u, s):
  return (u[0] + u[1]) / ((s[0] + s[1])[:, 0:1] + _EPS)


def _combine1_body(u0, s0, u1, s1, u2, s2, u3, s3, w2_ref, o_ref):
  h = jnp.concatenate(
      [_head_out(u0[...], s0[...]), _head_out(u1[...], s1[...]),
       _head_out(u2[...], s2[...]), _head_out(u3[...], s3[...])], axis=1)
  h = jnp.maximum(h, 0.0)
  o_ref[...] = jnp.dot(h, w2_ref[...], preferred_element_type=jnp.float32)


def _combine1(us, W2):
  grid = _N // _ROWBLK
  uspec = pl.BlockSpec((2, _ROWBLK, 64), lambda i: (0, i, 0))
  sspec = pl.BlockSpec((2, _ROWBLK, 16), lambda i: (0, i, 0))
  return pl.pallas_call(
      _combine1_body,
      grid=(grid,),
      in_specs=[uspec, sspec] * 4 + [pl.BlockSpec((256, 64), lambda i: (0, 0))],
      out_specs=pl.BlockSpec((_ROWBLK, 64), lambda i: (i, 0)),
      out_shape=jax.ShapeDtypeStruct((_N, 64), jnp.float32),
  )(*us, W2)


def _combine2_body(u_ref, s_ref, o_ref):
  o_ref[...] = _head_out(u_ref[...], s_ref[...])


def _combine2(uC, sC):
  grid = _N // _ROWBLK
  return pl.pallas_call(
      _combine2_body,
      grid=(grid,),
      in_specs=[
          pl.BlockSpec((2, _ROWBLK, 64), lambda i: (0, i, 0)),
          pl.BlockSpec((2, _ROWBLK, 16), lambda i: (0, i, 0)),
      ],
      out_specs=pl.BlockSpec((_ROWBLK, 64), lambda i: (i, 0)),
      out_shape=jax.ShapeDtypeStruct((_N, 64), jnp.float32),
  )(uC, sC)


def kernel(x, edge_index, W1, W2):
  src = edge_index[0].reshape(_NW, _NCHUNK, _K)
  dst = edge_index[1].reshape(_NW, _NCHUNK, _K)
  zu = jnp.zeros((_NP, _D), jnp.float32)
  zs = jnp.zeros((_NP, 16), jnp.float32)

  fts = _mm1(x, W1)
  us = []
  for ft in fts:
    u, s = _edge_pass(ft, src, dst, zu, zs)
    us += [u, s]
  ft2 = _combine1(us, W2)
  uC, sC = _edge_pass(ft2, src, dst, zu, zs)
  return _combine2(uC, sC)


# speedup vs baseline: 1.5020x; 1.5020x over previous
import functools

import jax
import jax.numpy as jnp
from jax import lax
from jax.experimental import pallas as pl
from jax.experimental.pallas import tpu as pltpu
from jax.experimental.pallas import tpu_sc as plsc

_N = 10000
_E = 320000
_D = 64
_SCALE = 1.0 / (_D ** 0.5)
_EPS = 1e-9

_NCORE = 2
_NSUB = 16
_NW = _NCORE * _NSUB
_EW = _E // _NW
_K = 100
_NCHUNK = _EW // _K
_NBUF = 2
_RPT = 632
_NP = _RPT * _NSUB

_ROWBLK = 1000


def _make_edge_pass():
  mesh = plsc.VectorSubcoreMesh(core_axis_name="c", subcore_axis_name="s",
                                num_cores=_NCORE, num_subcores=_NSUB)

  @functools.partial(
      pl.kernel,
      out_type=(
          jax.ShapeDtypeStruct((_NCORE, _NP, _D), jnp.float32),
          jax.ShapeDtypeStruct((_NCORE, _NP, 16), jnp.float32),
      ),
      mesh=mesh,
      scratch_types=[
          pltpu.VMEM_SHARED((_NP, _D), jnp.float32),
          pltpu.VMEM_SHARED((_NP, 16), jnp.float32),
          pltpu.VMEM((_NCHUNK, _K), jnp.int32),
          pltpu.VMEM((_NCHUNK, _K), jnp.int32),
          [pltpu.VMEM((_K, _D), jnp.float32)] * _NBUF,
          [pltpu.VMEM((_K, _D), jnp.float32)] * _NBUF,
          [pltpu.VMEM((_K, _D), jnp.float32)] * _NBUF,
          [pltpu.VMEM((_K, 16), jnp.float32)] * _NBUF,
          [pltpu.SemaphoreType.DMA] * _NBUF,
      ],
      compiler_params=pltpu.CompilerParams(use_tc_tiling_on_sc=False),
  )
  def kern(table_hbm, src_hbm, dst_hbm, zu_hbm, zs_hbm, u_hbm, s_hbm,
           u_sh, s_sh, sidx_v, didx_v, sbufs, dbufs, mbufs, wbufs, gsems):
    c = lax.axis_index("c")
    t = lax.axis_index("s")
    wid = c * _NSUB + t

    pltpu.sync_copy(src_hbm.at[wid], sidx_v)
    pltpu.sync_copy(dst_hbm.at[wid], didx_v)
    r0 = t * _RPT
    pltpu.sync_copy(zu_hbm.at[pl.ds(r0, _RPT)], u_sh.at[pl.ds(r0, _RPT)])
    pltpu.sync_copy(zs_hbm.at[pl.ds(r0, _RPT)], s_sh.at[pl.ds(r0, _RPT)])
    plsc.subcore_barrier()

    lanes = lax.iota(jnp.int32, 16)

    def allsum(v):
      for step in (8, 4, 2, 1):
        v = v + v.at[jnp.bitwise_xor(lanes, step)].get(
            mode="promise_in_bounds")
      return v

    def gather_issue(j, b):
      pltpu.async_copy(table_hbm.at[sidx_v.at[j]], sbufs[b], gsems[b])
      pltpu.async_copy(table_hbm.at[didx_v.at[j]], dbufs[b], gsems[b])

    def gather_wait(b):
      pltpu.make_async_copy(table_hbm.at[sidx_v.at[0]], sbufs[b],
                            gsems[b]).wait()
      pltpu.make_async_copy(table_hbm.at[didx_v.at[0]], dbufs[b],
                            gsems[b]).wait()

    def make_edge_body(sb, db, mb, wb):
      def edge_body(e):
        acc = jnp.zeros((16,), jnp.float32)
        for q in range(_D // 16):
          acc = acc + sb[e, pl.ds(q * 16, 16)] * db[e, pl.ds(q * 16, 16)]
        wv = jnp.exp(allsum(acc) * _SCALE)
        for q in range(_D // 16):
          mb[e, pl.ds(q * 16, 16)] = sb[e, pl.ds(q * 16, 16)] * wv
        wb[e] = jnp.where(lanes == 0, wv, 0.0)
      return edge_body

    for b in range(_NBUF):
      gather_issue(b, b)

    def step(s2, carry):
      for b in range(_NBUF):
        j = s2 * _NBUF + b
        gather_wait(b)
        plsc.parallel_loop(0, _K, 1, unroll=2)(
            make_edge_body(sbufs[b], dbufs[b], mbufs[b], wbufs[b]))
        idxd = didx_v.at[j]
        pltpu.sync_copy(mbufs[b], u_sh.at[idxd], add=True)
        pltpu.sync_copy(wbufs[b], s_sh.at[idxd], add=True)
        gather_issue(jnp.minimum(j + _NBUF, _NCHUNK - 1), b)
      return carry

    lax.fori_loop(0, _NCHUNK // _NBUF, step, 0)
    for b in range(_NBUF):
      gather_wait(b)

    plsc.subcore_barrier()
    pltpu.sync_copy(u_sh.at[pl.ds(r0, _RPT)], u_hbm.at[c, pl.ds(r0, _RPT)])
    pltpu.sync_copy(s_sh.at[pl.ds(r0, _RPT)], s_hbm.at[c, pl.ds(r0, _RPT)])

  return kern


_edge_pass = _make_edge_pass()


def _mm1_body(x_ref, w_ref, o0_ref, o1_ref, o2_ref, o3_ref):
  ft = jnp.dot(x_ref[...], w_ref[...], preferred_element_type=jnp.float32)
  o0_ref[...] = ft[:, 0:64]
  o1_ref[...] = ft[:, 64:128]
  o2_ref[...] = ft[:, 128:192]
  o3_ref[...] = ft[:, 192:256]


def _mm1(x, W1):
  grid = _N // _ROWBLK
  ospec = pl.BlockSpec((_ROWBLK, 64), lambda i: (i, 0))
  oshape = jax.ShapeDtypeStruct((_N, 64), jnp.float32)
  return pl.pallas_call(
      _mm1_body,
      grid=(grid,),
      in_specs=[
          pl.BlockSpec((_ROWBLK, 128), lambda i: (i, 0)),
          pl.BlockSpec((128, 256), lambda i: (0, 0)),
      ],
      out_specs=(ospec, ospec, ospec, ospec),
      out_shape=(oshape, oshape, oshape, oshape),
  )(x, W1)


def _head_out(u, s):
  return (u[0] + u[1]) / ((s[0] + s[1])[:, 0:1] + _EPS)


def _combine1_body(u0, s0, u1, s1, u2, s2, u3, s3, w2_ref, o_ref):
  h = jnp.concatenate(
      [_head_out(u0[...], s0[...]), _head_out(u1[...], s1[...]),
       _head_out(u2[...], s2[...]), _head_out(u3[...], s3[...])], axis=1)
  h = jnp.maximum(h, 0.0)
  o_ref[...] = jnp.dot(h, w2_ref[...], preferred_element_type=jnp.float32)


def _combine1(us, W2):
  grid = _N // _ROWBLK
  uspec = pl.BlockSpec((2, _ROWBLK, 64), lambda i: (0, i, 0))
  sspec = pl.BlockSpec((2, _ROWBLK, 16), lambda i: (0, i, 0))
  return pl.pallas_call(
      _combine1_body,
      grid=(grid,),
      in_specs=[uspec, sspec] * 4 + [pl.BlockSpec((256, 64), lambda i: (0, 0))],
      out_specs=pl.BlockSpec((_ROWBLK, 64), lambda i: (i, 0)),
      out_shape=jax.ShapeDtypeStruct((_N, 64), jnp.float32),
  )(*us, W2)


def _combine2_body(u_ref, s_ref, o_ref):
  o_ref[...] = _head_out(u_ref[...], s_ref[...])


def _combine2(uC, sC):
  grid = _N // _ROWBLK
  return pl.pallas_call(
      _combine2_body,
      grid=(grid,),
      in_specs=[
          pl.BlockSpec((2, _ROWBLK, 64), lambda i: (0, i, 0)),
          pl.BlockSpec((2, _ROWBLK, 16), lambda i: (0, i, 0)),
      ],
      out_specs=pl.BlockSpec((_ROWBLK, 64), lambda i: (i, 0)),
      out_shape=jax.ShapeDtypeStruct((_N, 64), jnp.float32),
  )(uC, sC)


def kernel(x, edge_index, W1, W2):
  src = edge_index[0].reshape(_NW, _NCHUNK, _K)
  dst = edge_index[1].reshape(_NW, _NCHUNK, _K)
  zu = jnp.zeros((_NP, _D), jnp.float32)
  zs = jnp.zeros((_NP, 16), jnp.float32)

  fts = _mm1(x, W1)
  us = []
  for ft in fts:
    u, s = _edge_pass(ft, src, dst, zu, zs)
    us += [u, s]
  ft2 = _combine1(us, W2)
  uC, sC = _edge_pass(ft2, src, dst, zu, zs)
  return _combine2(uC, sC)


# speedup vs baseline: 1.6430x; 1.0939x over previous
import functools

import jax
import jax.numpy as jnp
from jax import lax
from jax.experimental import pallas as pl
from jax.experimental.pallas import tpu as pltpu
from jax.experimental.pallas import tpu_sc as plsc

_N = 10000
_E = 320000
_D = 64
_SCALE = 1.0 / (_D ** 0.5)
_EPS = 1e-9

_NCORE = 2
_NSUB = 16
_NW = _NCORE * _NSUB
_EW = _E // _NW
_K = 100
_NCHUNK = _EW // _K
_NBUF = 2
_RPT = 632
_NP = _RPT * _NSUB

_ROWBLK = 1000


def _make_edge_pass():
  mesh = plsc.VectorSubcoreMesh(core_axis_name="c", subcore_axis_name="s",
                                num_cores=_NCORE, num_subcores=_NSUB)

  @functools.partial(
      pl.kernel,
      out_type=(
          jax.ShapeDtypeStruct((_NCORE, _NP, _D), jnp.float32),
          jax.ShapeDtypeStruct((_NCORE, _NP, 16), jnp.float32),
      ),
      mesh=mesh,
      scratch_types=[
          pltpu.VMEM_SHARED((_NP, _D), jnp.float32),
          pltpu.VMEM_SHARED((_NP, 16), jnp.float32),
          pltpu.VMEM((_NCHUNK, _K), jnp.int32),
          pltpu.VMEM((_NCHUNK, _K), jnp.int32),
          [pltpu.VMEM((_K, _D), jnp.float32)] * _NBUF,
          [pltpu.VMEM((_K, _D), jnp.float32)] * _NBUF,
          [pltpu.VMEM((_K, _D), jnp.float32)] * _NBUF,
          [pltpu.VMEM((_K, 16), jnp.float32)] * _NBUF,
          [pltpu.SemaphoreType.DMA] * _NBUF,
          [pltpu.SemaphoreType.DMA] * _NBUF,
      ],
      compiler_params=pltpu.CompilerParams(use_tc_tiling_on_sc=False),
  )
  def kern(table_hbm, src_hbm, dst_hbm, zu_hbm, zs_hbm, u_hbm, s_hbm,
           u_sh, s_sh, sidx_v, didx_v, sbufs, dbufs, mbufs, wbufs, gsems,
           ssems):
    c = lax.axis_index("c")
    t = lax.axis_index("s")
    wid = c * _NSUB + t

    pltpu.sync_copy(src_hbm.at[wid], sidx_v)
    pltpu.sync_copy(dst_hbm.at[wid], didx_v)
    r0 = t * _RPT
    pltpu.sync_copy(zu_hbm.at[pl.ds(r0, _RPT)], u_sh.at[pl.ds(r0, _RPT)])
    pltpu.sync_copy(zs_hbm.at[pl.ds(r0, _RPT)], s_sh.at[pl.ds(r0, _RPT)])
    plsc.subcore_barrier()

    lanes = lax.iota(jnp.int32, 16)

    def allsum(v):
      for step in (8, 4, 2, 1):
        v = v + v.at[jnp.bitwise_xor(lanes, step)].get(
            mode="promise_in_bounds")
      return v

    def gather_issue(j, b):
      pltpu.async_copy(table_hbm.at[sidx_v.at[j]], sbufs[b], gsems[b])
      pltpu.async_copy(table_hbm.at[didx_v.at[j]], dbufs[b], gsems[b])

    def gather_wait(b):
      pltpu.make_async_copy(table_hbm.at[sidx_v.at[0]], sbufs[b],
                            gsems[b]).wait()
      pltpu.make_async_copy(table_hbm.at[didx_v.at[0]], dbufs[b],
                            gsems[b]).wait()

    def scatter_issue(j, b):
      idxd = didx_v.at[j]
      pltpu.async_copy(mbufs[b], u_sh.at[idxd], ssems[b], add=True)
      pltpu.async_copy(wbufs[b], s_sh.at[idxd], ssems[b], add=True)

    def scatter_wait(b):
      pltpu.make_async_copy(mbufs[b], u_sh.at[didx_v.at[0]], ssems[b]).wait()
      pltpu.make_async_copy(wbufs[b], s_sh.at[didx_v.at[0]], ssems[b]).wait()

    def make_edge_body(sb, db, mb, wb):
      def edge_body(e):
        acc = jnp.zeros((16,), jnp.float32)
        for q in range(_D // 16):
          acc = acc + sb[e, pl.ds(q * 16, 16)] * db[e, pl.ds(q * 16, 16)]
        wv = jnp.exp(allsum(acc) * _SCALE)
        for q in range(_D // 16):
          mb[e, pl.ds(q * 16, 16)] = sb[e, pl.ds(q * 16, 16)] * wv
        wb[e] = jnp.where(lanes == 0, wv, 0.0)
      return edge_body

    for b in range(_NBUF):
      gather_issue(b, b)

    def step(s2, carry):
      for b in range(_NBUF):
        j = s2 * _NBUF + b
        gather_wait(b)

        @pl.when(s2 > 0)
        def _():
          scatter_wait(b)

        plsc.parallel_loop(0, _K, 1, unroll=2)(
            make_edge_body(sbufs[b], dbufs[b], mbufs[b], wbufs[b]))
        scatter_issue(j, b)
        gather_issue(jnp.minimum(j + _NBUF, _NCHUNK - 1), b)
      return carry

    lax.fori_loop(0, _NCHUNK // _NBUF, step, 0)
    for b in range(_NBUF):
      gather_wait(b)
      scatter_wait(b)

    plsc.subcore_barrier()
    pltpu.sync_copy(u_sh.at[pl.ds(r0, _RPT)], u_hbm.at[c, pl.ds(r0, _RPT)])
    pltpu.sync_copy(s_sh.at[pl.ds(r0, _RPT)], s_hbm.at[c, pl.ds(r0, _RPT)])

  return kern


_edge_pass = _make_edge_pass()


def _mm1_body(x_ref, w_ref, o0_ref, o1_ref, o2_ref, o3_ref):
  ft = jnp.dot(x_ref[...], w_ref[...], preferred_element_type=jnp.float32)
  o0_ref[...] = ft[:, 0:64]
  o1_ref[...] = ft[:, 64:128]
  o2_ref[...] = ft[:, 128:192]
  o3_ref[...] = ft[:, 192:256]


def _mm1(x, W1):
  grid = _N // _ROWBLK
  ospec = pl.BlockSpec((_ROWBLK, 64), lambda i: (i, 0))
  oshape = jax.ShapeDtypeStruct((_N, 64), jnp.float32)
  return pl.pallas_call(
      _mm1_body,
      grid=(grid,),
      in_specs=[
          pl.BlockSpec((_ROWBLK, 128), lambda i: (i, 0)),
          pl.BlockSpec((128, 256), lambda i: (0, 0)),
      ],
      out_specs=(ospec, ospec, ospec, ospec),
      out_shape=(oshape, oshape, oshape, oshape),
  )(x, W1)


def _head_out(u, s):
  return (u[0] + u[1]) / ((s[0] + s[1])[:, 0:1] + _EPS)


def _combine1_body(u0, s0, u1, s1, u2, s2, u3, s3, w2_ref, o_ref):
  h = jnp.concatenate(
      [_head_out(u0[...], s0[...]), _head_out(u1[...], s1[...]),
       _head_out(u2[...], s2[...]), _head_out(u3[...], s3[...])], axis=1)
  h = jnp.maximum(h, 0.0)
  o_ref[...] = jnp.dot(h, w2_ref[...], preferred_element_type=jnp.float32)


def _combine1(us, W2):
  grid = _N // _ROWBLK
  uspec = pl.BlockSpec((2, _ROWBLK, 64), lambda i: (0, i, 0))
  sspec = pl.BlockSpec((2, _ROWBLK, 16), lambda i: (0, i, 0))
  return pl.pallas_call(
      _combine1_body,
      grid=(grid,),
      in_specs=[uspec, sspec] * 4 + [pl.BlockSpec((256, 64), lambda i: (0, 0))],
      out_specs=pl.BlockSpec((_ROWBLK, 64), lambda i: (i, 0)),
      out_shape=jax.ShapeDtypeStruct((_N, 64), jnp.float32),
  )(*us, W2)


def _combine2_body(u_ref, s_ref, o_ref):
  o_ref[...] = _head_out(u_ref[...], s_ref[...])


def _combine2(uC, sC):
  grid = _N // _ROWBLK
  return pl.pallas_call(
      _combine2_body,
      grid=(grid,),
      in_specs=[
          pl.BlockSpec((2, _ROWBLK, 64), lambda i: (0, i, 0)),
          pl.BlockSpec((2, _ROWBLK, 16), lambda i: (0, i, 0)),
      ],
      out_specs=pl.BlockSpec((_ROWBLK, 64), lambda i: (i, 0)),
      out_shape=jax.ShapeDtypeStruct((_N, 64), jnp.float32),
  )(uC, sC)


def kernel(x, edge_index, W1, W2):
  src = edge_index[0].reshape(_NW, _NCHUNK, _K)
  dst = edge_index[1].reshape(_NW, _NCHUNK, _K)
  zu = jnp.zeros((_NP, _D), jnp.float32)
  zs = jnp.zeros((_NP, 16), jnp.float32)

  fts = _mm1(x, W1)
  us = []
  for ft in fts:
    u, s = _edge_pass(ft, src, dst, zu, zs)
    us += [u, s]
  ft2 = _combine1(us, W2)
  uC, sC = _edge_pass(ft2, src, dst, zu, zs)
  return _combine2(uC, sC)


# speedup vs baseline: 1.6529x; 1.0060x over previous
import functools

import jax
import jax.numpy as jnp
from jax import lax
from jax.experimental import pallas as pl
from jax.experimental.pallas import tpu as pltpu
from jax.experimental.pallas import tpu_sc as plsc

_N = 10000
_E = 320000
_D = 64
_SCALE = 1.0 / (_D ** 0.5)
_EPS = 1e-9

_NCORE = 2
_NSUB = 16
_NW = _NCORE * _NSUB
_EW = _E // _NW
_K = 125
_NCHUNK = _EW // _K
_NBUF = 2
_RPT = 632
_NP = _RPT * _NSUB

_ROWBLK = 1000


def _make_edge_pass():
  mesh = plsc.VectorSubcoreMesh(core_axis_name="c", subcore_axis_name="s",
                                num_cores=_NCORE, num_subcores=_NSUB)

  @functools.partial(
      pl.kernel,
      out_type=(
          jax.ShapeDtypeStruct((_NCORE, _NP, _D), jnp.float32),
          jax.ShapeDtypeStruct((_NCORE, _NP, 16), jnp.float32),
      ),
      mesh=mesh,
      scratch_types=[
          pltpu.VMEM_SHARED((_NP, _D), jnp.float32),
          pltpu.VMEM_SHARED((_NP, 16), jnp.float32),
          pltpu.VMEM((_NCHUNK, _K), jnp.int32),
          pltpu.VMEM((_NCHUNK, _K), jnp.int32),
          [pltpu.VMEM((_K, _D), jnp.float32)] * _NBUF,
          [pltpu.VMEM((_K, _D), jnp.float32)] * _NBUF,
          [pltpu.VMEM((_K, _D), jnp.float32)] * _NBUF,
          [pltpu.VMEM((_K, 16), jnp.float32)] * _NBUF,
          [pltpu.SemaphoreType.DMA] * _NBUF,
          [pltpu.SemaphoreType.DMA] * _NBUF,
      ],
      compiler_params=pltpu.CompilerParams(use_tc_tiling_on_sc=False),
  )
  def kern(table_hbm, src_hbm, dst_hbm, zu_hbm, zs_hbm, u_hbm, s_hbm,
           u_sh, s_sh, sidx_v, didx_v, sbufs, dbufs, mbufs, wbufs, gsems,
           ssems):
    c = lax.axis_index("c")
    t = lax.axis_index("s")
    wid = c * _NSUB + t

    pltpu.sync_copy(src_hbm.at[wid], sidx_v)
    pltpu.sync_copy(dst_hbm.at[wid], didx_v)
    r0 = t * _RPT
    pltpu.sync_copy(zu_hbm.at[pl.ds(r0, _RPT)], u_sh.at[pl.ds(r0, _RPT)])
    pltpu.sync_copy(zs_hbm.at[pl.ds(r0, _RPT)], s_sh.at[pl.ds(r0, _RPT)])
    plsc.subcore_barrier()

    lanes = lax.iota(jnp.int32, 16)

    def allsum(v):
      for step in (8, 4, 2, 1):
        v = v + v.at[jnp.bitwise_xor(lanes, step)].get(
            mode="promise_in_bounds")
      return v

    def gather_issue(j, b):
      pltpu.async_copy(table_hbm.at[sidx_v.at[j]], sbufs[b], gsems[b])
      pltpu.async_copy(table_hbm.at[didx_v.at[j]], dbufs[b], gsems[b])

    def gather_wait(b):
      pltpu.make_async_copy(table_hbm.at[sidx_v.at[0]], sbufs[b],
                            gsems[b]).wait()
      pltpu.make_async_copy(table_hbm.at[didx_v.at[0]], dbufs[b],
                            gsems[b]).wait()

    def scatter_issue(j, b):
      idxd = didx_v.at[j]
      pltpu.async_copy(mbufs[b], u_sh.at[idxd], ssems[b], add=True)
      pltpu.async_copy(wbufs[b], s_sh.at[idxd], ssems[b], add=True)

    def scatter_wait(b):
      pltpu.make_async_copy(mbufs[b], u_sh.at[didx_v.at[0]], ssems[b]).wait()
      pltpu.make_async_copy(wbufs[b], s_sh.at[didx_v.at[0]], ssems[b]).wait()

    def make_edge_body(sb, db, mb, wb):
      def edge_body(e):
        acc = jnp.zeros((16,), jnp.float32)
        for q in range(_D // 16):
          acc = acc + sb[e, pl.ds(q * 16, 16)] * db[e, pl.ds(q * 16, 16)]
        wv = jnp.exp(allsum(acc) * _SCALE)
        for q in range(_D // 16):
          mb[e, pl.ds(q * 16, 16)] = sb[e, pl.ds(q * 16, 16)] * wv
        wb[e] = jnp.where(lanes == 0, wv, 0.0)
      return edge_body

    for b in range(_NBUF):
      gather_issue(b, b)

    def step(s2, carry):
      for b in range(_NBUF):
        j = s2 * _NBUF + b
        gather_wait(b)

        @pl.when(s2 > 0)
        def _():
          scatter_wait(b)

        plsc.parallel_loop(0, _K, 1, unroll=2)(
            make_edge_body(sbufs[b], dbufs[b], mbufs[b], wbufs[b]))
        scatter_issue(j, b)
        gather_issue(jnp.minimum(j + _NBUF, _NCHUNK - 1), b)
      return carry

    lax.fori_loop(0, _NCHUNK // _NBUF, step, 0)
    for b in range(_NBUF):
      gather_wait(b)
      scatter_wait(b)

    plsc.subcore_barrier()
    pltpu.sync_copy(u_sh.at[pl.ds(r0, _RPT)], u_hbm.at[c, pl.ds(r0, _RPT)])
    pltpu.sync_copy(s_sh.at[pl.ds(r0, _RPT)], s_hbm.at[c, pl.ds(r0, _RPT)])

  return kern


_edge_pass = _make_edge_pass()


def _mm1_body(x_ref, w_ref, o0_ref, o1_ref, o2_ref, o3_ref):
  ft = jnp.dot(x_ref[...], w_ref[...], preferred_element_type=jnp.float32)
  o0_ref[...] = ft[:, 0:64]
  o1_ref[...] = ft[:, 64:128]
  o2_ref[...] = ft[:, 128:192]
  o3_ref[...] = ft[:, 192:256]


def _mm1(x, W1):
  grid = _N // _ROWBLK
  ospec = pl.BlockSpec((_ROWBLK, 64), lambda i: (i, 0))
  oshape = jax.ShapeDtypeStruct((_N, 64), jnp.float32)
  return pl.pallas_call(
      _mm1_body,
      grid=(grid,),
      in_specs=[
          pl.BlockSpec((_ROWBLK, 128), lambda i: (i, 0)),
          pl.BlockSpec((128, 256), lambda i: (0, 0)),
      ],
      out_specs=(ospec, ospec, ospec, ospec),
      out_shape=(oshape, oshape, oshape, oshape),
  )(x, W1)


def _head_out(u, s):
  return (u[0] + u[1]) / ((s[0] + s[1])[:, 0:1] + _EPS)


def _combine1_body(u0, s0, u1, s1, u2, s2, u3, s3, w2_ref, o_ref):
  h = jnp.concatenate(
      [_head_out(u0[...], s0[...]), _head_out(u1[...], s1[...]),
       _head_out(u2[...], s2[...]), _head_out(u3[...], s3[...])], axis=1)
  h = jnp.maximum(h, 0.0)
  o_ref[...] = jnp.dot(h, w2_ref[...], preferred_element_type=jnp.float32)


def _combine1(us, W2):
  grid = _N // _ROWBLK
  uspec = pl.BlockSpec((2, _ROWBLK, 64), lambda i: (0, i, 0))
  sspec = pl.BlockSpec((2, _ROWBLK, 16), lambda i: (0, i, 0))
  return pl.pallas_call(
      _combine1_body,
      grid=(grid,),
      in_specs=[uspec, sspec] * 4 + [pl.BlockSpec((256, 64), lambda i: (0, 0))],
      out_specs=pl.BlockSpec((_ROWBLK, 64), lambda i: (i, 0)),
      out_shape=jax.ShapeDtypeStruct((_N, 64), jnp.float32),
  )(*us, W2)


def _combine2_body(u_ref, s_ref, o_ref):
  o_ref[...] = _head_out(u_ref[...], s_ref[...])


def _combine2(uC, sC):
  grid = _N // _ROWBLK
  return pl.pallas_call(
      _combine2_body,
      grid=(grid,),
      in_specs=[
          pl.BlockSpec((2, _ROWBLK, 64), lambda i: (0, i, 0)),
          pl.BlockSpec((2, _ROWBLK, 16), lambda i: (0, i, 0)),
      ],
      out_specs=pl.BlockSpec((_ROWBLK, 64), lambda i: (i, 0)),
      out_shape=jax.ShapeDtypeStruct((_N, 64), jnp.float32),
  )(uC, sC)


def kernel(x, edge_index, W1, W2):
  src = edge_index[0].reshape(_NW, _NCHUNK, _K)
  dst = edge_index[1].reshape(_NW, _NCHUNK, _K)
  zu = jnp.zeros((_NP, _D), jnp.float32)
  zs = jnp.zeros((_NP, 16), jnp.float32)

  fts = _mm1(x, W1)
  us = []
  for ft in fts:
    u, s = _edge_pass(ft, src, dst, zu, zs)
    us += [u, s]
  ft2 = _combine1(us, W2)
  uC, sC = _edge_pass(ft2, src, dst, zu, zs)
  return _combine2(uC, sC)
